# baseline (speedup 1.0000x reference)
import functools

import jax
import jax.numpy as jnp
from jax.experimental import pallas as pl
from jax.experimental.pallas import tpu as pltpu


def _stack_rnn_kernel(n_iters, in_dim, x_ref, h0_ref, c0_ref, stack0_ref,
                      adw_ref, adb_ref, wcat_ref, bcat_ref,
                      outs_ref, hn_ref, cn_ref, stackout_ref,
                      stack_s, xi_s, sem_in, sem_out):
    k = pl.program_id(0)
    sd, sw, bsz = stack_s.shape
    hdim = hn_ref.shape[-1]
    hq = hdim // 4

    @pl.when(k == 0)
    def _init():
        hn_ref[...] = h0_ref[...]
        cn_ref[...] = c0_ref[...]
        cp = pltpu.make_async_copy(stack0_ref, stack_s, sem_in)
        cp.start()
        cp.wait()

    xi_s[:, in_dim + sw:] = hn_ref[0].astype(jnp.bfloat16)
    ad = jnp.dot(xi_s[:, in_dim + sw:], adw_ref[...],
                 preferred_element_type=jnp.float32) + adb_ref[...]
    d = jnp.tanh(ad[:, :sw])
    logits = ad[:, sw:]
    m = jnp.max(logits, axis=-1, keepdims=True)
    e = jnp.exp(logits - m)
    ctrl = e / jnp.sum(e, axis=-1, keepdims=True)
    ctrl_t = ctrl[:, :8].T
    a_push = ctrl_t[0:1, :]
    a_pop = ctrl_t[1:2, :]
    a_noop = ctrl_t[2:3, :]
    d_t = d.T

    xi_s[:, :in_dim] = x_ref[0]

    prev = d_t
    cur = stack_s[0]
    nxt = stack_s[1]
    top_t = a_noop * cur + a_push * prev + a_pop * nxt
    stack_s[0] = top_t
    xi_s[:, in_dim:in_dim + sw] = top_t.T.astype(jnp.bfloat16)
    prev = cur
    cur = nxt

    c = cn_ref[0]
    bcat = bcat_ref[...]

    for hc in range(4):
        g4 = jnp.dot(xi_s[...], wcat_ref[:, hc * 4 * hq:(hc + 1) * 4 * hq],
                     preferred_element_type=jnp.float32)
        g4 = g4 + bcat[:, hc * 4 * hq:(hc + 1) * 4 * hq]
        i_c = jax.nn.sigmoid(g4[:, 0 * hq:1 * hq])
        f_c = jax.nn.sigmoid(g4[:, 1 * hq:2 * hq])
        g_c = jnp.tanh(g4[:, 2 * hq:3 * hq])
        o_c = jax.nn.sigmoid(g4[:, 3 * hq:4 * hq])
        sl = slice(hc * hq, (hc + 1) * hq)
        c_c = f_c * c[:, sl] + i_c * g_c
        h_c = o_c * jnp.tanh(c_c)
        cn_ref[0, :, sl] = c_c
        hn_ref[0, :, sl] = h_c
        outs_ref[0, :, sl] = h_c

    for j in range(1, sd):
        nxt = stack_s[j + 1] if j < sd - 1 else None
        new_j = a_noop * cur + a_push * prev
        if nxt is not None:
            new_j = new_j + a_pop * nxt
        stack_s[j] = new_j
        prev = cur
        cur = nxt

    @pl.when(k == n_iters - 1)
    def _fin():
        cp = pltpu.make_async_copy(stack_s, stackout_ref, sem_out)
        cp.start()
        cp.wait()


def kernel(x, h0, c0, stack0, A_w, A_b, D_w, D_b, W_ih, W_hh, b_ih, b_hh):
    seq_len, bsz, in_dim = x.shape
    hdim = h0.shape[-1]
    sd, sw = stack0.shape[1], stack0.shape[2]
    n_ad = ((sw + 3 + 127) // 128) * 128
    kdim = in_dim + sw + hdim
    hq = hdim // 4

    adw = jnp.zeros((hdim, n_ad), jnp.float32)
    adw = adw.at[:, :sw].set(D_w.T).at[:, sw:sw + 3].set(A_w.T).astype(jnp.bfloat16)
    adb = jnp.full((1, n_ad), -1e9, jnp.float32)
    adb = adb.at[0, :sw].set(D_b).at[0, sw:sw + 3].set(A_b)
    wcat = jnp.concatenate([W_ih[:, :in_dim].T, W_ih[:, in_dim:].T, W_hh.T],
                           axis=0)
    wcat = (wcat.reshape(kdim, 4, 4, hq).transpose(0, 2, 1, 3)
            .reshape(kdim, 4 * hdim).astype(jnp.bfloat16))
    bias = (b_ih + b_hh).reshape(1, 4, 4, hq).transpose(0, 2, 1, 3).reshape(1, 4 * hdim)
    stack0_t = stack0.transpose(1, 2, 0)

    n_iters = seq_len
    outs, hn, cn, stack_t = pl.pallas_call(
        functools.partial(_stack_rnn_kernel, n_iters, in_dim),
        grid=(n_iters,),
        in_specs=[
            pl.BlockSpec((1, bsz, in_dim), lambda t: (t, 0, 0)),
            pl.BlockSpec((1, bsz, hdim), lambda t: (0, 0, 0)),
            pl.BlockSpec((1, bsz, hdim), lambda t: (0, 0, 0)),
            pl.BlockSpec(memory_space=pl.ANY),
            pl.BlockSpec((hdim, n_ad), lambda t: (0, 0)),
            pl.BlockSpec((1, n_ad), lambda t: (0, 0)),
            pl.BlockSpec((kdim, 4 * hdim), lambda t: (0, 0)),
            pl.BlockSpec((1, 4 * hdim), lambda t: (0, 0)),
        ],
        out_specs=[
            pl.BlockSpec((1, bsz, hdim), lambda t: (t, 0, 0)),
            pl.BlockSpec((1, bsz, hdim), lambda t: (0, 0, 0)),
            pl.BlockSpec((1, bsz, hdim), lambda t: (0, 0, 0)),
            pl.BlockSpec(memory_space=pl.ANY),
        ],
        out_shape=[
            jax.ShapeDtypeStruct((seq_len, bsz, hdim), jnp.float32),
            jax.ShapeDtypeStruct((1, bsz, hdim), jnp.float32),
            jax.ShapeDtypeStruct((1, bsz, hdim), jnp.float32),
            jax.ShapeDtypeStruct((sd, sw, bsz), jnp.float32),
        ],
        scratch_shapes=[
            pltpu.VMEM((sd, sw, bsz), jnp.float32),
            pltpu.VMEM((bsz, kdim), jnp.bfloat16),
            pltpu.SemaphoreType.DMA,
            pltpu.SemaphoreType.DMA,
        ],
        compiler_params=pltpu.CompilerParams(
            dimension_semantics=("arbitrary",),
            vmem_limit_bytes=48 * 1024 * 1024,
        ),
        name="stack_rnn",
    )(x.astype(jnp.bfloat16), h0, c0, stack0_t, adw, adb, wcat, bias)
    return outs, hn, cn, stack_t.transpose(2, 0, 1)


# speedup vs baseline: 1.1183x; 1.1183x over previous
import functools

import jax
import jax.numpy as jnp
from jax.experimental import pallas as pl
from jax.experimental.pallas import tpu as pltpu


def _stack_rnn_kernel(n_iters, t_block, x_ref, h0_ref, c0_ref, stack0_ref,
                      adw_ref, adb_ref, wx_ref, ws_ref, wh_ref, b_ref,
                      outs_ref, hn_ref, cn_ref, stackout_ref,
                      stack_s, sem_in, sem_out):
    k = pl.program_id(0)
    sd, bsz, sw = stack_s.shape
    hdim = hn_ref.shape[-1]

    @pl.when(k == 0)
    def _init():
        hn_ref[...] = h0_ref[...]
        cn_ref[...] = c0_ref[...]
        cp = pltpu.make_async_copy(stack0_ref, stack_s, sem_in)
        cp.start()
        cp.wait()

    h = hn_ref[0]
    c = cn_ref[0]

    for i in range(t_block):
        h_bf = h.astype(jnp.bfloat16)
        ad = jnp.dot(h_bf, adw_ref[...], preferred_element_type=jnp.float32) + adb_ref[...]
        d = jnp.tanh(ad[:, :sw])
        logits = ad[:, sw:]
        m = jnp.max(logits, axis=-1, keepdims=True)
        e = jnp.exp(logits - m)
        ctrl = e / jnp.sum(e, axis=-1, keepdims=True)
        a_push = ctrl[:, 0:1]
        a_pop = ctrl[:, 1:2]
        a_noop = ctrl[:, 2:3]

        top = None
        prev = d
        cur = stack_s[0]
        for j in range(sd):
            nxt = stack_s[j + 1] if j < sd - 1 else None
            new_j = a_noop * cur + a_push * prev
            if nxt is not None:
                new_j = new_j + a_pop * nxt
            stack_s[j] = new_j
            if j == 0:
                top = new_j
            prev = cur
            cur = nxt

        gates = (jnp.dot(x_ref[i], wx_ref[...], preferred_element_type=jnp.float32)
                 + jnp.dot(top.astype(jnp.bfloat16), ws_ref[...], preferred_element_type=jnp.float32)
                 + jnp.dot(h_bf, wh_ref[...], preferred_element_type=jnp.float32)
                 + b_ref[...])
        i_g = jax.nn.sigmoid(gates[:, :hdim])
        f_g = jax.nn.sigmoid(gates[:, hdim:2 * hdim])
        g_g = jnp.tanh(gates[:, 2 * hdim:3 * hdim])
        o_g = jax.nn.sigmoid(gates[:, 3 * hdim:])
        c = f_g * c + i_g * g_g
        h = o_g * jnp.tanh(c)
        outs_ref[i] = h

    cn_ref[0] = c
    hn_ref[0] = h

    @pl.when(k == n_iters - 1)
    def _fin():
        cp = pltpu.make_async_copy(stack_s, stackout_ref, sem_out)
        cp.start()
        cp.wait()


def kernel(x, h0, c0, stack0, A_w, A_b, D_w, D_b, W_ih, W_hh, b_ih, b_hh):
    seq_len, bsz, in_dim = x.shape
    hdim = h0.shape[-1]
    sd, sw = stack0.shape[1], stack0.shape[2]
    n_ad = ((sw + 3 + 127) // 128) * 128

    adw = jnp.zeros((hdim, n_ad), jnp.float32)
    adw = adw.at[:, :sw].set(D_w.T).at[:, sw:sw + 3].set(A_w.T).astype(jnp.bfloat16)
    adb = jnp.full((1, n_ad), -1e9, jnp.float32)
    adb = adb.at[0, :sw].set(D_b).at[0, sw:sw + 3].set(A_b)
    wx = W_ih[:, :in_dim].T.astype(jnp.bfloat16)
    ws = W_ih[:, in_dim:].T.astype(jnp.bfloat16)
    wh = W_hh.T.astype(jnp.bfloat16)
    bias = (b_ih + b_hh).reshape(1, 4 * hdim)
    stack0_t = stack0.transpose(1, 0, 2)

    t_block = 1
    n_iters = seq_len // t_block
    outs, hn, cn, stack_t = pl.pallas_call(
        functools.partial(_stack_rnn_kernel, n_iters, t_block),
        grid=(n_iters,),
        in_specs=[
            pl.BlockSpec((t_block, bsz, in_dim), lambda t: (t, 0, 0)),
            pl.BlockSpec((1, bsz, hdim), lambda t: (0, 0, 0)),
            pl.BlockSpec((1, bsz, hdim), lambda t: (0, 0, 0)),
            pl.BlockSpec(memory_space=pl.ANY),
            pl.BlockSpec((hdim, n_ad), lambda t: (0, 0)),
            pl.BlockSpec((1, n_ad), lambda t: (0, 0)),
            pl.BlockSpec((in_dim, 4 * hdim), lambda t: (0, 0)),
            pl.BlockSpec((sw, 4 * hdim), lambda t: (0, 0)),
            pl.BlockSpec((hdim, 4 * hdim), lambda t: (0, 0)),
            pl.BlockSpec((1, 4 * hdim), lambda t: (0, 0)),
        ],
        out_specs=[
            pl.BlockSpec((t_block, bsz, hdim), lambda t: (t, 0, 0)),
            pl.BlockSpec((1, bsz, hdim), lambda t: (0, 0, 0)),
            pl.BlockSpec((1, bsz, hdim), lambda t: (0, 0, 0)),
            pl.BlockSpec(memory_space=pl.ANY),
        ],
        out_shape=[
            jax.ShapeDtypeStruct((seq_len, bsz, hdim), jnp.float32),
            jax.ShapeDtypeStruct((1, bsz, hdim), jnp.float32),
            jax.ShapeDtypeStruct((1, bsz, hdim), jnp.float32),
            jax.ShapeDtypeStruct((sd, bsz, sw), jnp.float32),
        ],
        scratch_shapes=[
            pltpu.VMEM((sd, bsz, sw), jnp.float32),
            pltpu.SemaphoreType.DMA,
            pltpu.SemaphoreType.DMA,
        ],
        compiler_params=pltpu.CompilerParams(
            dimension_semantics=("arbitrary",),
            vmem_limit_bytes=48 * 1024 * 1024,
        ),
        name="stack_rnn",
    )(x.astype(jnp.bfloat16), h0, c0, stack0_t, adw, adb, wx, ws, wh, bias)
    return outs, hn, cn, stack_t.transpose(1, 0, 2)


# speedup vs baseline: 1.3313x; 1.1905x over previous
import functools

import jax
import jax.numpy as jnp
from jax.experimental import pallas as pl
from jax.experimental.pallas import tpu as pltpu


def _stack_rnn_kernel(n_iters, t_block, x_ref, h0_ref, c0_ref, stack0_ref,
                      adw_ref, adb_ref, wx_ref, ws_ref, wh_ref, b_ref,
                      outs_ref, hn_ref, cn_ref, stackout_ref,
                      stack_s, gxh_s, sem_in, sem_out):
    k = pl.program_id(0)
    sd, bsz, sw = stack_s.shape
    hdim = hn_ref.shape[-1]

    @pl.when(k == 0)
    def _init():
        hn_ref[...] = h0_ref[...]
        cn_ref[...] = c0_ref[...]
        cp = pltpu.make_async_copy(stack0_ref, stack_s, sem_in)
        cp.start()
        cp.wait()

    h = hn_ref[0]
    c = cn_ref[0]

    for i in range(t_block):
        h_bf = h.astype(jnp.bfloat16)
        ad = jnp.dot(h_bf, adw_ref[...], preferred_element_type=jnp.float32) + adb_ref[...]
        d = jnp.tanh(ad[:, :sw])
        logits = ad[:, sw:]
        m = jnp.max(logits, axis=-1, keepdims=True)
        e = jnp.exp(logits - m)
        ctrl = e / jnp.sum(e, axis=-1, keepdims=True)
        a_push = ctrl[:, 0:1]
        a_pop = ctrl[:, 1:2]
        a_noop = ctrl[:, 2:3]

        gxh_s[...] = (jnp.dot(x_ref[i], wx_ref[...], preferred_element_type=jnp.float32)
                      + jnp.dot(h_bf, wh_ref[...], preferred_element_type=jnp.float32)
                      + b_ref[...])

        top = None
        prev = d
        cur = stack_s[0]
        for j in range(sd):
            nxt = stack_s[j + 1] if j < sd - 1 else None
            new_j = a_noop * cur + a_push * prev
            if nxt is not None:
                new_j = new_j + a_pop * nxt
            stack_s[j] = new_j
            if j == 0:
                top = new_j
            prev = cur
            cur = nxt

        gates = (gxh_s[...]
                 + jnp.dot(top.astype(jnp.bfloat16), ws_ref[...],
                           preferred_element_type=jnp.float32))
        i_g = jax.nn.sigmoid(gates[:, :hdim])
        f_g = jax.nn.sigmoid(gates[:, hdim:2 * hdim])
        g_g = jnp.tanh(gates[:, 2 * hdim:3 * hdim])
        o_g = jax.nn.sigmoid(gates[:, 3 * hdim:])
        c = f_g * c + i_g * g_g
        h = o_g * jnp.tanh(c)
        outs_ref[i] = h

    cn_ref[0] = c
    hn_ref[0] = h

    @pl.when(k == n_iters - 1)
    def _fin():
        cp = pltpu.make_async_copy(stack_s, stackout_ref, sem_out)
        cp.start()
        cp.wait()


def kernel(x, h0, c0, stack0, A_w, A_b, D_w, D_b, W_ih, W_hh, b_ih, b_hh):
    seq_len, bsz, in_dim = x.shape
    hdim = h0.shape[-1]
    sd, sw = stack0.shape[1], stack0.shape[2]
    n_ad = ((sw + 3 + 127) // 128) * 128

    adw = jnp.zeros((hdim, n_ad), jnp.float32)
    adw = adw.at[:, :sw].set(D_w.T).at[:, sw:sw + 3].set(A_w.T).astype(jnp.bfloat16)
    adb = jnp.full((1, n_ad), -1e9, jnp.float32)
    adb = adb.at[0, :sw].set(D_b).at[0, sw:sw + 3].set(A_b)
    wx = W_ih[:, :in_dim].T.astype(jnp.bfloat16)
    ws = W_ih[:, in_dim:].T.astype(jnp.bfloat16)
    wh = W_hh.T.astype(jnp.bfloat16)
    bias = (b_ih + b_hh).reshape(1, 4 * hdim)
    stack0_t = stack0.transpose(1, 0, 2)

    t_block = 1
    n_iters = seq_len // t_block
    outs, hn, cn, stack_t = pl.pallas_call(
        functools.partial(_stack_rnn_kernel, n_iters, t_block),
        grid=(n_iters,),
        in_specs=[
            pl.BlockSpec((t_block, bsz, in_dim), lambda t: (t, 0, 0)),
            pl.BlockSpec((1, bsz, hdim), lambda t: (0, 0, 0)),
            pl.BlockSpec((1, bsz, hdim), lambda t: (0, 0, 0)),
            pl.BlockSpec(memory_space=pl.ANY),
            pl.BlockSpec((hdim, n_ad), lambda t: (0, 0)),
            pl.BlockSpec((1, n_ad), lambda t: (0, 0)),
            pl.BlockSpec((in_dim, 4 * hdim), lambda t: (0, 0)),
            pl.BlockSpec((sw, 4 * hdim), lambda t: (0, 0)),
            pl.BlockSpec((hdim, 4 * hdim), lambda t: (0, 0)),
            pl.BlockSpec((1, 4 * hdim), lambda t: (0, 0)),
        ],
        out_specs=[
            pl.BlockSpec((t_block, bsz, hdim), lambda t: (t, 0, 0)),
            pl.BlockSpec((1, bsz, hdim), lambda t: (0, 0, 0)),
            pl.BlockSpec((1, bsz, hdim), lambda t: (0, 0, 0)),
            pl.BlockSpec(memory_space=pl.ANY),
        ],
        out_shape=[
            jax.ShapeDtypeStruct((seq_len, bsz, hdim), jnp.float32),
            jax.ShapeDtypeStruct((1, bsz, hdim), jnp.float32),
            jax.ShapeDtypeStruct((1, bsz, hdim), jnp.float32),
            jax.ShapeDtypeStruct((sd, bsz, sw), jnp.float32),
        ],
        scratch_shapes=[
            pltpu.VMEM((sd, bsz, sw), jnp.float32),
            pltpu.VMEM((bsz, 4 * hdim), jnp.float32),
            pltpu.SemaphoreType.DMA,
            pltpu.SemaphoreType.DMA,
        ],
        compiler_params=pltpu.CompilerParams(
            dimension_semantics=("arbitrary",),
            vmem_limit_bytes=48 * 1024 * 1024,
        ),
        name="stack_rnn",
    )(x.astype(jnp.bfloat16), h0, c0, stack0_t, adw, adb, wx, ws, wh, bias)
    return outs, hn, cn, stack_t.transpose(1, 0, 2)
